# Initial kernel scaffold; baseline (speedup 1.0000x reference)
import jax
import jax.numpy as jnp
from jax.experimental import pallas as pl


def kernel(y_pred, y_true, time):
    raise NotImplementedError("write your pallas kernel here")


# speedup vs baseline: 1.8683x; 1.8683x over previous
import functools

import jax
import jax.numpy as jnp
from jax import lax
from jax.experimental import pallas as pl

LANES = 128


def _partner(x, d, ri, ci, rows):
    if d >= rows:
        dc = d // rows
        bit = (ci & dc) != 0
        return jnp.where(bit, jnp.roll(x, dc, axis=1), jnp.roll(x, -dc, axis=1))
    else:
        bit = (ri & d) != 0
        return jnp.where(bit, jnp.roll(x, d, axis=0), jnp.roll(x, -d, axis=0))


def _cox_kernel(rows, logn, t_ref, e_ref, lh_ref, out_ref):
    t = t_ref[...]
    e = e_ref[...]
    lh = lh_ref[...]

    ri = lax.broadcasted_iota(jnp.int32, (rows, LANES), 0)
    ci = lax.broadcasted_iota(jnp.int32, (rows, LANES), 1)
    idx = ci * rows + ri

    lmax = jnp.max(lh)
    hr = jnp.exp(lh - lmax)
    sum_e = jnp.sum(e)
    sum_elh = jnp.sum(e * lh)

    k, ev, h = t, e, hr
    for st in range(logn):
        up = ((idx >> (st + 1)) & 1) == 0
        for sub in range(st, -1, -1):
            d = 1 << sub
            pk = _partner(k, d, ri, ci, rows)
            pe = _partner(ev, d, ri, ci, rows)
            ph = _partner(h, d, ri, ci, rows)
            j_less = (idx & d) == 0
            keep_min = j_less == up
            take = (keep_min & (pk < k)) | (~keep_min & (pk > k))
            k = jnp.where(take, pk, k)
            ev = jnp.where(take, pe, ev)
            h = jnp.where(take, ph, h)

    li = lax.broadcasted_iota(jnp.int32, (rows, rows), 0)
    lj = lax.broadcasted_iota(jnp.int32, (rows, rows), 1)
    lower = jnp.where(lj <= li, 1.0, 0.0).astype(jnp.float32)
    csum = jnp.dot(lower, h, preferred_element_type=jnp.float32)
    colsum = csum[rows - 1 : rows, :]
    ui = lax.broadcasted_iota(jnp.int32, (LANES, LANES), 0)
    uj = lax.broadcasted_iota(jnp.int32, (LANES, LANES), 1)
    strict_upper = jnp.where(ui < uj, 1.0, 0.0).astype(jnp.float32)
    off = jnp.dot(colsum, strict_upper, preferred_element_type=jnp.float32)
    cum = csum + off

    term = jnp.sum(ev * jnp.log(cum + 1e-6))
    neg_likelihood = -(sum_elh - term - lmax * sum_e)
    loss = neg_likelihood / (sum_e + 1e-6)
    out_ref[...] = jnp.broadcast_to(loss, (1, 1))


@functools.partial(jax.jit, static_argnums=())
def _cox_loss(t, e, lh):
    n = t.shape[0]
    rows = n // LANES
    logn = n.bit_length() - 1
    f = pl.pallas_call(
        functools.partial(_cox_kernel, rows, logn),
        out_shape=jax.ShapeDtypeStruct((1, 1), jnp.float32),
    )
    return f(
        t.reshape(rows, LANES), e.reshape(rows, LANES), lh.reshape(rows, LANES)
    )[0, 0]


def kernel(y_pred, y_true, time):
    return _cox_loss(
        time.reshape(-1).astype(jnp.float32),
        y_true.reshape(-1).astype(jnp.float32),
        y_pred.reshape(-1).astype(jnp.float32),
    )
